# baseline (speedup 1.0000x reference)
from itertools import combinations

import jax
import jax.numpy as jnp
import numpy as np
from jax import lax
from jax.experimental import pallas as pl
from jax.experimental.pallas import tpu as pltpu
from jax.experimental.pallas import tpu_sc as plsc

_N_MODES = 20
_N_PHOTONS = 5
_N_OUT_MODES = 10

_B = 1024
_NIN = 15504
_NOUT = 252
_K = 32
_L = 16
_NS = 16

_B_SC = 256
_B_TC = _B - _B_SC
_NB = 2
_BW = 128
_NQ = 16
_OPW = 16
_TAIL = _NOUT - (_NQ - 1) * _OPW
_RPW = _OPW * _K
_CHO = 4
_CHR = _CHO * _K
_NCHUNK = _OPW // _CHO
_SEG = _BW // _L
_TCB = 256


def _build_tables():
    def fock_keys(n_modes, n_photons):
        ks = []
        for comb in combinations(range(n_modes), n_photons):
            occ = [0] * n_modes
            for m in comb:
                occ[m] = 1
            ks.append(tuple(occ))
        return ks

    keys_in = fock_keys(_N_MODES, _N_PHOTONS)
    keys_out = fock_keys(_N_OUT_MODES, _N_PHOTONS)
    num_skips = _N_MODES // _N_OUT_MODES
    first_skips = _N_MODES % _N_OUT_MODES
    index_num_skips = list(range(0, _N_MODES + 1, num_skips))
    index_first_skips = ([0] + list(range(1, first_skips + 1))
                         + [first_skips] * (_N_OUT_MODES - first_skips))
    skips = [a + b for a, b in zip(index_first_skips, index_num_skips)]
    groups = [list(range(skips[k], skips[k + 1])) for k in range(_N_OUT_MODES)]
    out_index = {k: i for i, k in enumerate(keys_out)}
    match, include = [], []
    for i, kin in enumerate(keys_in):
        kout = tuple(sum(kin[m] for m in g) for g in groups)
        if kout in out_index:
            match.append(out_index[kout])
            include.append(i)
    match = np.asarray(match, np.int64)
    include = np.asarray(include, np.int64)
    order = np.argsort(match, kind="stable")
    grouped = include[order].reshape(_NOUT, _K)
    padded = np.zeros((_NQ * _OPW, _K), np.int64)
    padded[:_NOUT] = grouped
    idx = padded.reshape(-1).astype(np.int32)
    pool = np.zeros((_NIN, 256), np.float32)
    pool[include, match] = 1.0
    return idx, pool


_IDX, _POOL = _build_tables()


def _rsqrt16(x):
    xi = plsc.bitcast(x, jnp.int32)
    yi = jnp.int32(0x5F3759DF) - lax.shift_right_arithmetic(xi, 1)
    y = plsc.bitcast(yi, jnp.float32)
    for _ in range(3):
        y = y * (jnp.float32(1.5) - jnp.float32(0.5) * x * y * y)
    return y


def _compute_chunk(buf, outst, c):
    def seg_body(s, carry):
        col = s * _L
        for j in range(_CHO):
            acc = buf[j * _K, pl.ds(col, _L)]
            for k in range(1, _K):
                acc = acc + buf[j * _K + k, pl.ds(col, _L)]
            outst[c * _CHO + j, pl.ds(col, _L)] = acc
        return carry

    lax.fori_loop(0, _SEG, seg_body, 0)


def _sc_body(at, idxt, outt, idx_v, buf_a, buf_b, outst, nrm, shared,
             sem_a, sem_b):
    cid = lax.axis_index("c")
    sid = lax.axis_index("s")
    nb = cid
    q = sid
    col0 = _B_TC + nb * _BW
    o0 = q * _OPW

    pltpu.sync_copy(idxt.at[pl.ds(q * _RPW, _RPW)], idx_v)
    pltpu.async_copy(at.at[idx_v.at[pl.ds(0, _CHR)], pl.ds(col0, _BW)],
                     buf_a, sem_a)

    def pair_body(i, carry):
        c0 = 2 * i
        pltpu.make_async_copy(at.at[idx_v.at[pl.ds(0, _CHR)],
                                    pl.ds(col0, _BW)], buf_a, sem_a).wait()
        pltpu.async_copy(at.at[idx_v.at[pl.ds((c0 + 1) * _CHR, _CHR)],
                               pl.ds(col0, _BW)], buf_b, sem_b)
        _compute_chunk(buf_a, outst, c0)

        pltpu.make_async_copy(at.at[idx_v.at[pl.ds(0, _CHR)],
                                    pl.ds(col0, _BW)], buf_b, sem_b).wait()
        nxt = jnp.minimum((c0 + 2) * _CHR, jnp.int32((_NCHUNK - 1) * _CHR))
        pltpu.async_copy(at.at[idx_v.at[pl.ds(nxt, _CHR)],
                               pl.ds(col0, _BW)], buf_a, sem_a)
        _compute_chunk(buf_b, outst, c0 + 1)
        return carry

    lax.fori_loop(0, _NCHUNK // 2, pair_body, 0)
    pltpu.make_async_copy(at.at[idx_v.at[pl.ds(0, _CHR)],
                                pl.ds(col0, _BW)], buf_a, sem_a).wait()

    opw = jnp.where(q == _NQ - 1, _TAIL, _OPW)
    for s in range(_SEG):
        col = s * _L

        def ssq_body(r, ssq):
            v = outst[r, pl.ds(col, _L)]
            return ssq + v * v

        nrm[0, pl.ds(col, _L)] = lax.fori_loop(
            0, opw, ssq_body, jnp.zeros((_L,), jnp.float32))

    pltpu.sync_copy(nrm.at[0], shared.at[sid])
    plsc.subcore_barrier()
    g0 = (sid // _NQ) * _NQ
    for p in range(_NQ):
        pltpu.sync_copy(shared.at[g0 + p], nrm.at[1 + p])
    for s in range(_SEG):
        col = s * _L
        tot = nrm[1, pl.ds(col, _L)]
        for p in range(1, _NQ):
            tot = tot + nrm[1 + p, pl.ds(col, _L)]
        nrm[0, pl.ds(col, _L)] = _rsqrt16(tot)

    def scale_body(r, carry):
        for s in range(_SEG):
            col = s * _L
            outst[r, pl.ds(col, _L)] = (outst[r, pl.ds(col, _L)]
                                        * nrm[0, pl.ds(col, _L)])
        return carry

    lax.fori_loop(0, _OPW, scale_body, 0)

    @pl.when(q < _NQ - 1)
    def _():
        pltpu.sync_copy(outst,
                        outt.at[pl.ds(o0, _OPW), pl.ds(nb * _BW, _BW)])

    @pl.when(q == _NQ - 1)
    def _():
        pltpu.sync_copy(outst.at[pl.ds(0, _TAIL)],
                        outt.at[pl.ds((_NQ - 1) * _OPW, _TAIL),
                                pl.ds(nb * _BW, _BW)])


def _tc_body(at_ref, pool_ref, out_ref):
    a = at_ref[...]
    hi = a.astype(jnp.bfloat16)
    lo = (a - hi.astype(jnp.float32)).astype(jnp.bfloat16)
    dn = (((0,), (0,)), ((), ()))
    p = pool_ref[...]
    m = (jax.lax.dot_general(hi, p, dn, preferred_element_type=jnp.float32)
         + jax.lax.dot_general(lo, p, dn, preferred_element_type=jnp.float32))
    ssq = jnp.sum(m * m, axis=1, keepdims=True)
    out_ref[...] = (m * jax.lax.rsqrt(ssq))[:, :_NOUT]


def kernel(amplitudes):
    at = amplitudes.T
    idxt = jnp.asarray(_IDX)
    pool = jnp.asarray(_POOL, dtype=jnp.bfloat16)

    mesh = plsc.VectorSubcoreMesh(core_axis_name="c", subcore_axis_name="s")
    sc_run = pl.kernel(
        _sc_body,
        out_type=jax.ShapeDtypeStruct((_NOUT, _B_SC), jnp.float32),
        mesh=mesh,
        compiler_params=pltpu.CompilerParams(use_tc_tiling_on_sc=True,
                                             needs_layout_passes=False),
        scratch_types=[
            pltpu.VMEM((_RPW,), jnp.int32),
            pltpu.VMEM((_CHR, _BW), jnp.float32),
            pltpu.VMEM((_CHR, _BW), jnp.float32),
            pltpu.VMEM((_OPW, _BW), jnp.float32),
            pltpu.VMEM((1 + _NQ, _BW), jnp.float32),
            pltpu.VMEM_SHARED((_NS, _BW), jnp.float32),
            pltpu.SemaphoreType.DMA,
            pltpu.SemaphoreType.DMA,
        ],
    )
    out_sc_t = sc_run(at, idxt)

    tc_run = pl.pallas_call(
        _tc_body,
        out_shape=jax.ShapeDtypeStruct((_B_TC, _NOUT), jnp.float32),
        grid=(_B_TC // _TCB,),
        in_specs=[
            pl.BlockSpec((_NIN, _TCB), lambda i: (0, i)),
            pl.BlockSpec((_NIN, 256), lambda i: (0, 0)),
        ],
        out_specs=pl.BlockSpec((_TCB, _NOUT), lambda i: (i, 0)),
    )
    out_tc = tc_run(at, pool)

    return jnp.concatenate([out_tc, out_sc_t.T], axis=0)


# speedup vs baseline: 1.0034x; 1.0034x over previous
from itertools import combinations

import jax
import jax.numpy as jnp
import numpy as np
from jax import lax
from jax.experimental import pallas as pl
from jax.experimental.pallas import tpu as pltpu
from jax.experimental.pallas import tpu_sc as plsc

_N_MODES = 20
_N_PHOTONS = 5
_N_OUT_MODES = 10

_B = 1024
_NIN = 15504
_NOUT = 252
_K = 32
_L = 16
_NS = 16

_B_SC = 256
_B_TC = _B - _B_SC
_NB = 2
_BW = 128
_NQ = 16
_OPW = 16
_TAIL = _NOUT - (_NQ - 1) * _OPW
_RPW = _OPW * _K
_CHO = 4
_CHR = _CHO * _K
_NCHUNK = _OPW // _CHO
_SEG = _BW // _L
_TCB = 128


def _build_tables():
    def fock_keys(n_modes, n_photons):
        ks = []
        for comb in combinations(range(n_modes), n_photons):
            occ = [0] * n_modes
            for m in comb:
                occ[m] = 1
            ks.append(tuple(occ))
        return ks

    keys_in = fock_keys(_N_MODES, _N_PHOTONS)
    keys_out = fock_keys(_N_OUT_MODES, _N_PHOTONS)
    num_skips = _N_MODES // _N_OUT_MODES
    first_skips = _N_MODES % _N_OUT_MODES
    index_num_skips = list(range(0, _N_MODES + 1, num_skips))
    index_first_skips = ([0] + list(range(1, first_skips + 1))
                         + [first_skips] * (_N_OUT_MODES - first_skips))
    skips = [a + b for a, b in zip(index_first_skips, index_num_skips)]
    groups = [list(range(skips[k], skips[k + 1])) for k in range(_N_OUT_MODES)]
    out_index = {k: i for i, k in enumerate(keys_out)}
    match, include = [], []
    for i, kin in enumerate(keys_in):
        kout = tuple(sum(kin[m] for m in g) for g in groups)
        if kout in out_index:
            match.append(out_index[kout])
            include.append(i)
    match = np.asarray(match, np.int64)
    include = np.asarray(include, np.int64)
    order = np.argsort(match, kind="stable")
    grouped = include[order].reshape(_NOUT, _K)
    padded = np.zeros((_NQ * _OPW, _K), np.int64)
    padded[:_NOUT] = grouped
    idx = padded.reshape(-1).astype(np.int32)
    pool = np.zeros((_NIN, 256), np.float32)
    pool[include, match] = 1.0
    return idx, pool


_IDX, _POOL = _build_tables()


def _rsqrt16(x):
    xi = plsc.bitcast(x, jnp.int32)
    yi = jnp.int32(0x5F3759DF) - lax.shift_right_arithmetic(xi, 1)
    y = plsc.bitcast(yi, jnp.float32)
    for _ in range(3):
        y = y * (jnp.float32(1.5) - jnp.float32(0.5) * x * y * y)
    return y


def _compute_chunk(buf, outst, c):
    def seg_body(s, carry):
        col = s * _L
        for j in range(_CHO):
            acc = buf[j * _K, pl.ds(col, _L)]
            for k in range(1, _K):
                acc = acc + buf[j * _K + k, pl.ds(col, _L)]
            outst[c * _CHO + j, pl.ds(col, _L)] = acc
        return carry

    lax.fori_loop(0, _SEG, seg_body, 0)


def _sc_body(at, idxt, outt, idx_v, buf_a, buf_b, outst, nrm, shared,
             sem_a, sem_b):
    cid = lax.axis_index("c")
    sid = lax.axis_index("s")
    nb = cid
    q = sid
    col0 = _B_TC + nb * _BW
    o0 = q * _OPW

    pltpu.sync_copy(idxt.at[pl.ds(q * _RPW, _RPW)], idx_v)
    pltpu.async_copy(at.at[idx_v.at[pl.ds(0, _CHR)], pl.ds(col0, _BW)],
                     buf_a, sem_a)

    def pair_body(i, carry):
        c0 = 2 * i
        pltpu.make_async_copy(at.at[idx_v.at[pl.ds(0, _CHR)],
                                    pl.ds(col0, _BW)], buf_a, sem_a).wait()
        pltpu.async_copy(at.at[idx_v.at[pl.ds((c0 + 1) * _CHR, _CHR)],
                               pl.ds(col0, _BW)], buf_b, sem_b)
        _compute_chunk(buf_a, outst, c0)

        pltpu.make_async_copy(at.at[idx_v.at[pl.ds(0, _CHR)],
                                    pl.ds(col0, _BW)], buf_b, sem_b).wait()
        nxt = jnp.minimum((c0 + 2) * _CHR, jnp.int32((_NCHUNK - 1) * _CHR))
        pltpu.async_copy(at.at[idx_v.at[pl.ds(nxt, _CHR)],
                               pl.ds(col0, _BW)], buf_a, sem_a)
        _compute_chunk(buf_b, outst, c0 + 1)
        return carry

    lax.fori_loop(0, _NCHUNK // 2, pair_body, 0)
    pltpu.make_async_copy(at.at[idx_v.at[pl.ds(0, _CHR)],
                                pl.ds(col0, _BW)], buf_a, sem_a).wait()

    opw = jnp.where(q == _NQ - 1, _TAIL, _OPW)
    for s in range(_SEG):
        col = s * _L

        def ssq_body(r, ssq):
            v = outst[r, pl.ds(col, _L)]
            return ssq + v * v

        nrm[0, pl.ds(col, _L)] = lax.fori_loop(
            0, opw, ssq_body, jnp.zeros((_L,), jnp.float32))

    pltpu.sync_copy(nrm.at[0], shared.at[sid])
    plsc.subcore_barrier()
    g0 = (sid // _NQ) * _NQ
    for p in range(_NQ):
        pltpu.sync_copy(shared.at[g0 + p], nrm.at[1 + p])
    for s in range(_SEG):
        col = s * _L
        tot = nrm[1, pl.ds(col, _L)]
        for p in range(1, _NQ):
            tot = tot + nrm[1 + p, pl.ds(col, _L)]
        nrm[0, pl.ds(col, _L)] = _rsqrt16(tot)

    def scale_body(r, carry):
        for s in range(_SEG):
            col = s * _L
            outst[r, pl.ds(col, _L)] = (outst[r, pl.ds(col, _L)]
                                        * nrm[0, pl.ds(col, _L)])
        return carry

    lax.fori_loop(0, _OPW, scale_body, 0)

    @pl.when(q < _NQ - 1)
    def _():
        pltpu.sync_copy(outst,
                        outt.at[pl.ds(o0, _OPW), pl.ds(nb * _BW, _BW)])

    @pl.when(q == _NQ - 1)
    def _():
        pltpu.sync_copy(outst.at[pl.ds(0, _TAIL)],
                        outt.at[pl.ds((_NQ - 1) * _OPW, _TAIL),
                                pl.ds(nb * _BW, _BW)])


def _tc_body(at_ref, pool_ref, out_ref):
    a = at_ref[...]
    hi = a.astype(jnp.bfloat16)
    lo = (a - hi.astype(jnp.float32)).astype(jnp.bfloat16)
    dn = (((0,), (0,)), ((), ()))
    p = pool_ref[...]
    m = (jax.lax.dot_general(hi, p, dn, preferred_element_type=jnp.float32)
         + jax.lax.dot_general(lo, p, dn, preferred_element_type=jnp.float32))
    ssq = jnp.sum(m * m, axis=1, keepdims=True)
    out_ref[...] = (m * jax.lax.rsqrt(ssq))[:, :_NOUT]


def kernel(amplitudes):
    at = amplitudes.T
    idxt = jnp.asarray(_IDX)
    pool = jnp.asarray(_POOL, dtype=jnp.bfloat16)

    mesh = plsc.VectorSubcoreMesh(core_axis_name="c", subcore_axis_name="s")
    sc_run = pl.kernel(
        _sc_body,
        out_type=jax.ShapeDtypeStruct((_NOUT, _B_SC), jnp.float32),
        mesh=mesh,
        compiler_params=pltpu.CompilerParams(use_tc_tiling_on_sc=True,
                                             needs_layout_passes=False),
        scratch_types=[
            pltpu.VMEM((_RPW,), jnp.int32),
            pltpu.VMEM((_CHR, _BW), jnp.float32),
            pltpu.VMEM((_CHR, _BW), jnp.float32),
            pltpu.VMEM((_OPW, _BW), jnp.float32),
            pltpu.VMEM((1 + _NQ, _BW), jnp.float32),
            pltpu.VMEM_SHARED((_NS, _BW), jnp.float32),
            pltpu.SemaphoreType.DMA,
            pltpu.SemaphoreType.DMA,
        ],
    )
    out_sc_t = sc_run(at, idxt)

    tc_run = pl.pallas_call(
        _tc_body,
        out_shape=jax.ShapeDtypeStruct((_B_TC, _NOUT), jnp.float32),
        grid=(_B_TC // _TCB,),
        in_specs=[
            pl.BlockSpec((_NIN, _TCB), lambda i: (0, i)),
            pl.BlockSpec((_NIN, 256), lambda i: (0, 0)),
        ],
        out_specs=pl.BlockSpec((_TCB, _NOUT), lambda i: (i, 0)),
    )
    out_tc = tc_run(at, pool)

    return jnp.concatenate([out_tc, out_sc_t.T], axis=0)
